# baseline (speedup 1.0000x reference)
import functools

import jax
import jax.numpy as jnp
from jax import lax
from jax.experimental import pallas as pl
from jax.experimental.pallas import tpu as pltpu
from jax.experimental.pallas import tpu_sc as plsc

N = 10000
E = 320000
D = 128
DH = D // 2
G = 64
C = 10

NW = 32
TILES = 16
CHUNK = 128
CPT0 = 60
CPT1 = 100
E_PAD = TILES * (CPT0 + CPT1) * CHUNK
N_PAD = 10112
RPT = N_PAD // TILES
NH_PAD = 10240
NPT = NH_PAD // TILES
DUMMY = N + 8

RB = 2000
NB = N // RB


@functools.cache
def _make_agg():
    mesh = plsc.VectorSubcoreMesh(core_axis_name="c", subcore_axis_name="s",
                                  num_cores=2, num_subcores=TILES)

    @functools.partial(
        pl.kernel,
        out_type=jax.ShapeDtypeStruct((2 * N_PAD, D), jnp.float32),
        mesh=mesh,
        scratch_types=[
            pltpu.VMEM((CHUNK,), jnp.int32),
            pltpu.VMEM((CHUNK,), jnp.int32),
            pltpu.VMEM((CHUNK,), jnp.int32),
            pltpu.VMEM((CHUNK,), jnp.int32),
            pltpu.VMEM((CHUNK, D), jnp.float32),
            pltpu.VMEM((CHUNK, D), jnp.float32),
            pltpu.VMEM_SHARED((N_PAD, D), jnp.float32),
            pltpu.SemaphoreType.DMA,
            pltpu.SemaphoreType.DMA,
            pltpu.SemaphoreType.DMA,
            pltpu.SemaphoreType.DMA,
        ],
    )
    def agg(h_hbm, src_hbm, dst_hbm, zeros_hbm, out_hbm,
            src_v0, src_v1, dst_v0, dst_v1, rows0, rows1, acc_sh,
            sg0, sg1, ss0, ss1):
        c = lax.axis_index("c")
        s = lax.axis_index("s")

        pltpu.sync_copy(zeros_hbm, acc_sh.at[pl.ds(s * RPT, RPT)])
        plsc.subcore_barrier()

        def make_body(base0):
            def body(i, carry):
                base = base0 + i * CHUNK
                pltpu.sync_copy(src_hbm.at[pl.ds(base, CHUNK)], src_v0)
                d0 = pltpu.async_copy(h_hbm.at[src_v0], rows0, sg0)
                pltpu.sync_copy(dst_hbm.at[pl.ds(base, CHUNK)], dst_v0)
                d0.wait()
                pltpu.sync_copy(rows0, acc_sh.at[dst_v0], add=True)
                return carry
            return body

        @pl.when(c == 0)
        def _():
            lax.fori_loop(0, CPT0, make_body(s * (CPT0 * CHUNK)), 0)

        @pl.when(c == 1)
        def _():
            lax.fori_loop(
                0, CPT1,
                make_body((TILES * CPT0 + s * CPT1) * CHUNK), 0)
        plsc.subcore_barrier()

        pltpu.sync_copy(acc_sh.at[pl.ds(s * RPT, RPT)],
                        out_hbm.at[pl.ds(c * N_PAD + s * RPT, RPT)])

    return agg


def _dense_a_body(p0_ref, p1_ref, h_ref, wr_ref, wo_ref, br_ref,
                  y_ref, stats_ref):
    i = pl.program_id(0)
    p = p0_ref[0] + p1_ref[0]
    h = h_ref[...]
    y = jnp.dot(p, wr_ref[...], preferred_element_type=jnp.float32)
    y = y + jnp.dot(h, wo_ref[...], preferred_element_type=jnp.float32)
    y = y + br_ref[...]
    y_ref[...] = y

    @pl.when(i == 0)
    def _():
        stats_ref[...] = jnp.zeros_like(stats_ref)

    stats_ref[0:1, :] += jnp.sum(y, axis=0, keepdims=True)
    stats_ref[1:2, :] += jnp.sum(y * y, axis=0, keepdims=True)


_dense_a = pl.pallas_call(
    _dense_a_body,
    grid=(NB,),
    in_specs=[
        pl.BlockSpec((1, RB, D), lambda i: (0, i, 0)),
        pl.BlockSpec((1, RB, D), lambda i: (1, i, 0)),
        pl.BlockSpec((RB, D), lambda i: (i, 0)),
        pl.BlockSpec((D, D), lambda i: (0, 0)),
        pl.BlockSpec((D, D), lambda i: (0, 0)),
        pl.BlockSpec((1, D), lambda i: (0, 0)),
    ],
    out_specs=[
        pl.BlockSpec((RB, D), lambda i: (i, 0)),
        pl.BlockSpec((2, D), lambda i: (0, 0)),
    ],
    out_shape=[
        jax.ShapeDtypeStruct((N, D), jnp.float32),
        jax.ShapeDtypeStruct((2, D), jnp.float32),
    ],
)


def _dense_b_body(y_ref, stats_ref, g_ref, b_ref, o_ref):
    mean = stats_ref[0:1, :] / N
    var = stats_ref[1:2, :] / N - mean * mean
    inv = lax.rsqrt(var + 1e-5)
    o_ref[...] = jnp.maximum(
        (y_ref[...] - mean) * inv * g_ref[...] + b_ref[...], 0.0)


_dense_b = pl.pallas_call(
    _dense_b_body,
    grid=(NB,),
    in_specs=[
        pl.BlockSpec((RB, D), lambda i: (i, 0)),
        pl.BlockSpec((2, D), lambda i: (0, 0)),
        pl.BlockSpec((1, D), lambda i: (0, 0)),
        pl.BlockSpec((1, D), lambda i: (0, 0)),
    ],
    out_specs=pl.BlockSpec((RB, D), lambda i: (i, 0)),
    out_shape=jax.ShapeDtypeStruct((N, D), jnp.float32),
)


def _pool_body(h_ref, batch_ref, wc_ref, bc_ref, o_ref,
               acc_ref, cnt_ref):
    i = pl.program_id(0)

    @pl.when(i == 0)
    def _():
        acc_ref[...] = jnp.zeros_like(acc_ref)
        cnt_ref[...] = jnp.zeros_like(cnt_ref)

    b = batch_ref[0]
    h = h_ref[...]
    gids = lax.broadcasted_iota(jnp.int32, (G, RB), 0)
    oh = (gids == b).astype(jnp.float32)
    acc_ref[...] += jnp.dot(oh, h, preferred_element_type=jnp.float32)
    cnt_ref[...] += jnp.sum(oh, axis=1, keepdims=True)

    @pl.when(i == NB - 1)
    def _():
        pooled = acc_ref[...] / jnp.clip(cnt_ref[...], 1.0, None)
        o_ref[...] = (jnp.dot(pooled, wc_ref[...],
                              preferred_element_type=jnp.float32)
                      + bc_ref[...])


_pool = pl.pallas_call(
    _pool_body,
    grid=(NB,),
    in_specs=[
        pl.BlockSpec((RB, D), lambda i: (i, 0)),
        pl.BlockSpec((1, 1, RB), lambda i: (i, 0, 0)),
        pl.BlockSpec((D, D), lambda i: (0, 0)),
        pl.BlockSpec((1, D), lambda i: (0, 0)),
    ],
    out_specs=pl.BlockSpec((G, D), lambda i: (0, 0)),
    out_shape=jax.ShapeDtypeStruct((G, D), jnp.float32),
    scratch_shapes=[
        pltpu.VMEM((G, D), jnp.float32),
        pltpu.VMEM((G, 1), jnp.float32),
    ],
)


def kernel(x, edge_index, batch,
           W_rel0, b_rel0, W_root0, gamma0, beta0,
           W_rel1, b_rel1, W_root1, gamma1, beta1,
           W_rel2, b_rel2, W_root2, gamma2, beta2,
           W_cls, b_cls):
    pad = E_PAD - E
    src = jnp.concatenate([edge_index[0], jnp.zeros((pad,), jnp.int32)])
    dst = jnp.concatenate([edge_index[1], jnp.full((pad,), DUMMY, jnp.int32)])
    zeros = jnp.zeros((RPT, D), jnp.float32)
    batch3 = batch.reshape(NB, 1, RB)

    layers = [
        (W_rel0, b_rel0, W_root0, gamma0, beta0),
        (W_rel1, b_rel1, W_root1, gamma1, beta1),
        (W_rel2, b_rel2, W_root2, gamma2, beta2),
    ]
    agg_fn = _make_agg()
    h = x
    for (Wr, br, Wo, g, b) in layers:
        parts = agg_fn(h, src, dst, zeros).reshape(2, N_PAD, D)
        y, stats = _dense_a(parts, parts, h, Wr, Wo, br.reshape(1, D))
        h = _dense_b(y, stats, g.reshape(1, D), b.reshape(1, D))

    wc = jnp.zeros((D, D), jnp.float32).at[:, :C].set(W_cls)
    bc = jnp.zeros((1, D), jnp.float32).at[0, :C].set(b_cls)
    out = _pool(h, batch3, wc, bc)
    return out[:, :C]


# speedup vs baseline: 1.6381x; 1.6381x over previous
import functools

import jax
import jax.numpy as jnp
from jax import lax
from jax.experimental import pallas as pl
from jax.experimental.pallas import tpu as pltpu
from jax.experimental.pallas import tpu_sc as plsc

N = 10000
E = 320000
D = 128
DH = D // 2
G = 64
C = 10

NW = 32
TILES = 16
CHUNK = 128
CHUNKS_PER_TILE = 79
E_PAD = CHUNKS_PER_TILE * NW * CHUNK
N_PAD = 10112
RPT = N_PAD // TILES
NH_PAD = 10240
NPT = NH_PAD // TILES
DUMMY = N + 8

RB = 2000
NB = N // RB


@functools.cache
def _make_agg():
    mesh = plsc.VectorSubcoreMesh(core_axis_name="c", subcore_axis_name="s",
                                  num_cores=2, num_subcores=TILES)

    @functools.partial(
        pl.kernel,
        out_type=jax.ShapeDtypeStruct((2 * N_PAD, D), jnp.float32),
        mesh=mesh,
        scratch_types=[
            pltpu.VMEM((CHUNK,), jnp.int32),
            pltpu.VMEM((CHUNK,), jnp.int32),
            pltpu.VMEM((CHUNK,), jnp.int32),
            pltpu.VMEM((CHUNK,), jnp.int32),
            pltpu.VMEM((CHUNK, D), jnp.float32),
            pltpu.VMEM((CHUNK, D), jnp.float32),
            pltpu.VMEM_SHARED((N_PAD, D), jnp.float32),
            pltpu.SemaphoreType.DMA,
            pltpu.SemaphoreType.DMA,
            pltpu.SemaphoreType.DMA,
            pltpu.SemaphoreType.DMA,
        ],
    )
    def agg(h_hbm, src_hbm, dst_hbm, zeros_hbm, out_hbm,
            src_v0, src_v1, dst_v0, dst_v1, rows0, rows1, acc_sh,
            sg0, sg1, ss0, ss1):
        c = lax.axis_index("c")
        s = lax.axis_index("s")
        wid = c * TILES + s
        base0 = wid * (CHUNKS_PER_TILE * CHUNK)

        pltpu.sync_copy(zeros_hbm, acc_sh.at[pl.ds(s * RPT, RPT)])
        plsc.subcore_barrier()

        def body(i, carry):
            base = base0 + i * CHUNK
            pltpu.sync_copy(src_hbm.at[pl.ds(base, CHUNK)], src_v0)
            d0 = pltpu.async_copy(h_hbm.at[src_v0], rows0, sg0)
            pltpu.sync_copy(dst_hbm.at[pl.ds(base, CHUNK)], dst_v0)
            d0.wait()
            pltpu.sync_copy(rows0, acc_sh.at[dst_v0], add=True)
            return carry

        lax.fori_loop(0, CHUNKS_PER_TILE, body, 0)
        plsc.subcore_barrier()

        pltpu.sync_copy(acc_sh.at[pl.ds(s * RPT, RPT)],
                        out_hbm.at[pl.ds(c * N_PAD + s * RPT, RPT)])

    return agg


def _dense_a_body(p0_ref, p1_ref, h_ref, wr_ref, wo_ref, br_ref,
                  y_ref, stats_ref):
    i = pl.program_id(0)
    p = p0_ref[0] + p1_ref[0]
    h = h_ref[...]
    y = jnp.dot(p, wr_ref[...], preferred_element_type=jnp.float32)
    y = y + jnp.dot(h, wo_ref[...], preferred_element_type=jnp.float32)
    y = y + br_ref[...]
    y_ref[...] = y

    @pl.when(i == 0)
    def _():
        stats_ref[...] = jnp.zeros_like(stats_ref)

    stats_ref[0:1, :] += jnp.sum(y, axis=0, keepdims=True)
    stats_ref[1:2, :] += jnp.sum(y * y, axis=0, keepdims=True)


_dense_a = pl.pallas_call(
    _dense_a_body,
    grid=(NB,),
    in_specs=[
        pl.BlockSpec((1, RB, D), lambda i: (0, i, 0)),
        pl.BlockSpec((1, RB, D), lambda i: (1, i, 0)),
        pl.BlockSpec((RB, D), lambda i: (i, 0)),
        pl.BlockSpec((D, D), lambda i: (0, 0)),
        pl.BlockSpec((D, D), lambda i: (0, 0)),
        pl.BlockSpec((1, D), lambda i: (0, 0)),
    ],
    out_specs=[
        pl.BlockSpec((RB, D), lambda i: (i, 0)),
        pl.BlockSpec((2, D), lambda i: (0, 0)),
    ],
    out_shape=[
        jax.ShapeDtypeStruct((N, D), jnp.float32),
        jax.ShapeDtypeStruct((2, D), jnp.float32),
    ],
)


def _dense_b_body(y_ref, stats_ref, g_ref, b_ref, o_ref):
    mean = stats_ref[0:1, :] / N
    var = stats_ref[1:2, :] / N - mean * mean
    inv = lax.rsqrt(var + 1e-5)
    o_ref[...] = jnp.maximum(
        (y_ref[...] - mean) * inv * g_ref[...] + b_ref[...], 0.0)


_dense_b = pl.pallas_call(
    _dense_b_body,
    grid=(NB,),
    in_specs=[
        pl.BlockSpec((RB, D), lambda i: (i, 0)),
        pl.BlockSpec((2, D), lambda i: (0, 0)),
        pl.BlockSpec((1, D), lambda i: (0, 0)),
        pl.BlockSpec((1, D), lambda i: (0, 0)),
    ],
    out_specs=pl.BlockSpec((RB, D), lambda i: (i, 0)),
    out_shape=jax.ShapeDtypeStruct((N, D), jnp.float32),
)


def _pool_body(h_ref, batch_ref, wc_ref, bc_ref, o_ref,
               acc_ref, cnt_ref):
    i = pl.program_id(0)

    @pl.when(i == 0)
    def _():
        acc_ref[...] = jnp.zeros_like(acc_ref)
        cnt_ref[...] = jnp.zeros_like(cnt_ref)

    b = batch_ref[0]
    h = h_ref[...]
    gids = lax.broadcasted_iota(jnp.int32, (G, RB), 0)
    oh = (gids == b).astype(jnp.float32)
    acc_ref[...] += jnp.dot(oh, h, preferred_element_type=jnp.float32)
    cnt_ref[...] += jnp.sum(oh, axis=1, keepdims=True)

    @pl.when(i == NB - 1)
    def _():
        pooled = acc_ref[...] / jnp.clip(cnt_ref[...], 1.0, None)
        o_ref[...] = (jnp.dot(pooled, wc_ref[...],
                              preferred_element_type=jnp.float32)
                      + bc_ref[...])


_pool = pl.pallas_call(
    _pool_body,
    grid=(NB,),
    in_specs=[
        pl.BlockSpec((RB, D), lambda i: (i, 0)),
        pl.BlockSpec((1, 1, RB), lambda i: (i, 0, 0)),
        pl.BlockSpec((D, D), lambda i: (0, 0)),
        pl.BlockSpec((1, D), lambda i: (0, 0)),
    ],
    out_specs=pl.BlockSpec((G, D), lambda i: (0, 0)),
    out_shape=jax.ShapeDtypeStruct((G, D), jnp.float32),
    scratch_shapes=[
        pltpu.VMEM((G, D), jnp.float32),
        pltpu.VMEM((G, 1), jnp.float32),
    ],
)


def kernel(x, edge_index, batch,
           W_rel0, b_rel0, W_root0, gamma0, beta0,
           W_rel1, b_rel1, W_root1, gamma1, beta1,
           W_rel2, b_rel2, W_root2, gamma2, beta2,
           W_cls, b_cls):
    pad = E_PAD - E
    src = jnp.concatenate([edge_index[0], jnp.zeros((pad,), jnp.int32)])
    dst = jnp.concatenate([edge_index[1], jnp.full((pad,), DUMMY, jnp.int32)])
    zeros = jnp.zeros((RPT, D), jnp.float32)
    batch3 = batch.reshape(NB, 1, RB)

    layers = [
        (W_rel0, b_rel0, W_root0, gamma0, beta0),
        (W_rel1, b_rel1, W_root1, gamma1, beta1),
        (W_rel2, b_rel2, W_root2, gamma2, beta2),
    ]
    agg_fn = _make_agg()
    h = x
    for (Wr, br, Wo, g, b) in layers:
        parts = agg_fn(h, src, dst, zeros).reshape(2, N_PAD, D)
        y, stats = _dense_a(parts, parts, h, Wr, Wo, br.reshape(1, D))
        h = _dense_b(y, stats, g.reshape(1, D), b.reshape(1, D))

    wc = jnp.zeros((D, D), jnp.float32).at[:, :C].set(W_cls)
    bc = jnp.zeros((1, D), jnp.float32).at[0, :C].set(b_cls)
    out = _pool(h, batch3, wc, bc)
    return out[:, :C]
